# baseline (speedup 1.0000x reference)
import jax
import jax.numpy as jnp
from jax import lax
from jax.experimental import pallas as pl
from jax.experimental.pallas import tpu as pltpu
from jax.experimental.pallas import tpu_sc as plsc

NUM_EMB = 1000000
DIM = 32
PAD_IDX = 0
TOTAL = 4096 * 200
NC, NS, L = 2, 16, 16
NW = NC * NS

RBLK = 128
NFULL = NUM_EMB // RBLK
TAIL = NUM_EMB - NFULL * RBLK


def _tr_body(tbl_t, tail_lin, lin_hbm, t0, t1, l0, l1, sin0, sin1, so0, so1):
    wid = lax.axis_index("s") * NC + lax.axis_index("c")
    iv_base = lax.broadcasted_iota(jnp.int32, (L,), 0) * DIM

    def transpose_block(tile_v, lin_v, nrows):
        for d in range(DIM):
            for c in range(nrows // L):
                v = tile_v[d, pl.ds(c * L, L)]
                plsc.store_scatter(lin_v, [iv_base + (c * L * DIM + d)], v)

    bufs = ((t0, l0, sin0, so0), (t1, l1, sin1, so1))
    nsteps = 2 * (((NFULL - 1) // NW + 1 + 1) // 2)

    def blk(i):
        return (wid + i * NW) * RBLK

    def active(i):
        return blk(i) < NFULL * RBLK

    def start_in(i, parity):
        tile_v, _, sin, _ = bufs[parity]

        @pl.when(active(i))
        def _():
            pltpu.async_copy(tbl_t.at[:, pl.ds(blk(i), RBLK)], tile_v, sin)

    def step(i, parity, tile_v, lin_v, sin, so):
        start_in(i + 1, 1 - parity)

        @pl.when((i >= 2) & active(i - 2))
        def _():
            pltpu.make_async_copy(
                lin_v, lin_hbm.at[pl.ds(0, RBLK * DIM)], so
            ).wait()

        @pl.when(active(i))
        def _():
            pltpu.make_async_copy(
                tbl_t.at[:, pl.ds(blk(i), RBLK)], tile_v, sin
            ).wait()
            transpose_block(tile_v, lin_v, RBLK)
            pltpu.async_copy(
                lin_v, lin_hbm.at[pl.ds(blk(i) * DIM, RBLK * DIM)], so
            )

    start_in(0, 0)

    def pair_body(k, carry):
        for b in range(2):
            tile_v, lin_v, sin, so = bufs[b]
            step(2 * k + b, b, tile_v, lin_v, sin, so)
        return carry

    lax.fori_loop(0, nsteps // 2, pair_body, 0)
    for b in range(2):
        tile_v, lin_v, sin, so = bufs[b]
        last_i = nsteps - 2 + b
        @pl.when(active(last_i))
        def _():
            pltpu.make_async_copy(
                lin_v, lin_hbm.at[pl.ds(0, RBLK * DIM)], so
            ).wait()

    @pl.when(wid == 0)
    def _():
        pltpu.sync_copy(tail_lin, l0.at[pl.ds(0, TAIL * DIM)])
        pltpu.sync_copy(
            l0.at[pl.ds(0, TAIL * DIM)],
            lin_hbm.at[pl.ds(NFULL * RBLK * DIM, TAIL * DIM)],
        )


def _transpose_table(embeddings_t, tail_lin):
    mesh = plsc.VectorSubcoreMesh(core_axis_name="c", subcore_axis_name="s")
    return pl.kernel(
        _tr_body,
        out_type=jax.ShapeDtypeStruct((NUM_EMB * DIM,), jnp.float32),
        mesh=mesh,
        compiler_params=pltpu.CompilerParams(
            use_tc_tiling_on_sc=True, needs_layout_passes=False),
        scratch_types=[
            pltpu.VMEM((DIM, RBLK), jnp.float32),
            pltpu.VMEM((DIM, RBLK), jnp.float32),
            pltpu.VMEM((RBLK * DIM,), jnp.float32),
            pltpu.VMEM((RBLK * DIM,), jnp.float32),
            pltpu.SemaphoreType.DMA,
            pltpu.SemaphoreType.DMA,
            pltpu.SemaphoreType.DMA,
            pltpu.SemaphoreType.DMA,
        ],
    )(embeddings_t, tail_lin)


ROWS_PER_W = TOTAL // NW
CHUNK = 1280
NCHUNK = ROWS_PER_W // CHUNK
GROUPS = CHUNK // L


def _body(idx_hbm, table_hbm, out_hbm, idx_v, rows0, rows1, sg0, sg1, ss0, ss1):
    wid = lax.axis_index("s") * NC + lax.axis_index("c")
    wbase = wid * ROWS_PER_W
    pltpu.sync_copy(idx_hbm.at[pl.ds(wbase, ROWS_PER_W)], idx_v)

    def pad_scan(ci):
        def scan_body(g, acc):
            return jnp.minimum(acc, idx_v[pl.ds(ci * CHUNK + g * L, L)])

        acc = lax.fori_loop(
            0, GROUPS, scan_body, jnp.full((L,), NUM_EMB, jnp.int32)
        )
        mn = acc[0]
        for i in range(1, L):
            mn = jnp.minimum(mn, acc[i])
        return mn

    def fixup(ci, mn, rows_v):
        @pl.when(mn == PAD_IDX)
        def _():
            z = jnp.zeros((L,), jnp.float32)

            def fix_body(g, c):
                v = idx_v[pl.ds(ci * CHUNK + g * L, L)]
                for r in range(L):
                    @pl.when(v[r] == PAD_IDX)
                    def _zero_row(row=g * L + r):
                        for h in range(DIM // L):
                            rows_v[row, pl.ds(h * L, L)] = z

                return c

            lax.fori_loop(0, GROUPS, fix_body, 0)

    bufs = ((rows0, sg0, ss0), (rows1, sg1, ss1))

    def do_chunk(ci, rows_v, sg, ss):
        @pl.when(ci >= 2)
        def _():
            pltpu.make_async_copy(
                rows_v, out_hbm.at[pl.ds(wbase, CHUNK)], ss
            ).wait()

        gather = pltpu.async_copy(
            table_hbm.at[idx_v.at[pl.ds(ci * CHUNK, CHUNK)]], rows_v, sg
        )
        mn = pad_scan(ci)
        gather.wait()
        fixup(ci, mn, rows_v)
        pltpu.async_copy(
            rows_v, out_hbm.at[pl.ds(wbase + ci * CHUNK, CHUNK)], ss
        )

    def pair_body(k, carry):
        for b in range(2):
            rows_v, sg, ss = bufs[b]
            do_chunk(2 * k + b, rows_v, sg, ss)
        return carry

    lax.fori_loop(0, NCHUNK // 2, pair_body, 0)

    for b in range(2):
        rows_v, _, ss = bufs[b]
        pltpu.make_async_copy(
            rows_v, out_hbm.at[pl.ds(wbase, CHUNK)], ss
        ).wait()


def kernel(idx, embeddings, padding_mult):
    tail_lin = embeddings[NFULL * RBLK:].reshape(-1)
    lin_flat = _transpose_table(embeddings.T, tail_lin)
    lin_table = lin_flat.reshape(NUM_EMB, DIM)
    idx_flat = idx.reshape(-1)
    mesh = plsc.VectorSubcoreMesh(core_axis_name="c", subcore_axis_name="s")
    out = pl.kernel(
        _body,
        out_type=jax.ShapeDtypeStruct((TOTAL, DIM), jnp.float32),
        mesh=mesh,
        compiler_params=pltpu.CompilerParams(use_tc_tiling_on_sc=False),
        scratch_types=[
            pltpu.VMEM((ROWS_PER_W,), jnp.int32),
            pltpu.VMEM((CHUNK, DIM), jnp.float32),
            pltpu.VMEM((CHUNK, DIM), jnp.float32),
            pltpu.SemaphoreType.DMA,
            pltpu.SemaphoreType.DMA,
            pltpu.SemaphoreType.DMA,
            pltpu.SemaphoreType.DMA,
        ],
    )(idx_flat, lin_table)
    return out.reshape(idx.shape + (DIM,))


# speedup vs baseline: 1.4594x; 1.4594x over previous
import jax
import jax.numpy as jnp
from jax import lax
from jax.experimental import pallas as pl
from jax.experimental.pallas import tpu as pltpu
from jax.experimental.pallas import tpu_sc as plsc

NUM_EMB = 1000000
DIM = 32
PAD_IDX = 0
TOTAL = 4096 * 200
NC, NS, L = 2, 16, 16
NW = NC * NS

RBLK = 128
NFULL = NUM_EMB // RBLK
TAIL = NUM_EMB - NFULL * RBLK


def _tr_body(tbl_t, tail_lin, lin_hbm, t0, t1, l0, l1, sin0, sin1, so0, so1):
    wid = lax.axis_index("s") * NC + lax.axis_index("c")
    iota = lax.broadcasted_iota(jnp.int32, (L,), 0)

    def transpose_block(tile_v, lin_v, nrows):
        nsub = (DIM // L) * (nrows // L)

        def sub_body(t, carry):
            d_base = (t >> 3) * L
            colv = (t & 7) * L + iota
            for k in range(L):
                rowv = d_base + ((iota + k) & (L - 1))
                v = plsc.load_gather(tile_v, [rowv, colv])
                plsc.store_scatter(lin_v, [colv * DIM + rowv], v)
            return carry

        lax.fori_loop(0, nsub, sub_body, 0)

    bufs = ((t0, l0, sin0, so0), (t1, l1, sin1, so1))
    nsteps = 2 * (((NFULL - 1) // NW + 1 + 1) // 2)

    def blk(i):
        return (wid + i * NW) * RBLK

    def active(i):
        return blk(i) < NFULL * RBLK

    def start_in(i, parity):
        tile_v, _, sin, _ = bufs[parity]

        @pl.when(active(i))
        def _():
            pltpu.async_copy(tbl_t.at[:, pl.ds(blk(i), RBLK)], tile_v, sin)

    def step(i, parity, tile_v, lin_v, sin, so):
        start_in(i + 1, 1 - parity)

        @pl.when((i >= 2) & active(i - 2))
        def _():
            pltpu.make_async_copy(
                lin_v, lin_hbm.at[pl.ds(0, RBLK * DIM)], so
            ).wait()

        @pl.when(active(i))
        def _():
            pltpu.make_async_copy(
                tbl_t.at[:, pl.ds(blk(i), RBLK)], tile_v, sin
            ).wait()
            transpose_block(tile_v, lin_v, RBLK)
            pltpu.async_copy(
                lin_v, lin_hbm.at[pl.ds(blk(i) * DIM, RBLK * DIM)], so
            )

    start_in(0, 0)

    def pair_body(k, carry):
        for b in range(2):
            tile_v, lin_v, sin, so = bufs[b]
            step(2 * k + b, b, tile_v, lin_v, sin, so)
        return carry

    lax.fori_loop(0, nsteps // 2, pair_body, 0)
    for b in range(2):
        tile_v, lin_v, sin, so = bufs[b]
        last_i = nsteps - 2 + b
        @pl.when(active(last_i))
        def _():
            pltpu.make_async_copy(
                lin_v, lin_hbm.at[pl.ds(0, RBLK * DIM)], so
            ).wait()

    @pl.when(wid == 0)
    def _():
        pltpu.sync_copy(tail_lin, l0.at[pl.ds(0, TAIL * DIM)])
        pltpu.sync_copy(
            l0.at[pl.ds(0, TAIL * DIM)],
            lin_hbm.at[pl.ds(NFULL * RBLK * DIM, TAIL * DIM)],
        )


def _transpose_table(embeddings_t, tail_lin):
    mesh = plsc.VectorSubcoreMesh(core_axis_name="c", subcore_axis_name="s")
    return pl.kernel(
        _tr_body,
        out_type=jax.ShapeDtypeStruct((NUM_EMB * DIM,), jnp.float32),
        mesh=mesh,
        compiler_params=pltpu.CompilerParams(
            use_tc_tiling_on_sc=True, needs_layout_passes=False),
        scratch_types=[
            pltpu.VMEM((DIM, RBLK), jnp.float32),
            pltpu.VMEM((DIM, RBLK), jnp.float32),
            pltpu.VMEM((RBLK * DIM,), jnp.float32),
            pltpu.VMEM((RBLK * DIM,), jnp.float32),
            pltpu.SemaphoreType.DMA,
            pltpu.SemaphoreType.DMA,
            pltpu.SemaphoreType.DMA,
            pltpu.SemaphoreType.DMA,
        ],
    )(embeddings_t, tail_lin)


ROWS_PER_W = TOTAL // NW
CHUNK = 1280
NCHUNK = ROWS_PER_W // CHUNK
GROUPS = CHUNK // L


def _body(idx_hbm, table_hbm, out_hbm, idx_v, rows0, rows1, sg0, sg1, ss0, ss1):
    wid = lax.axis_index("s") * NC + lax.axis_index("c")
    wbase = wid * ROWS_PER_W
    pltpu.sync_copy(idx_hbm.at[pl.ds(wbase, ROWS_PER_W)], idx_v)

    def pad_scan(ci):
        def scan_body(g, acc):
            return jnp.minimum(acc, idx_v[pl.ds(ci * CHUNK + g * L, L)])

        acc = lax.fori_loop(
            0, GROUPS, scan_body, jnp.full((L,), NUM_EMB, jnp.int32)
        )
        mn = acc[0]
        for i in range(1, L):
            mn = jnp.minimum(mn, acc[i])
        return mn

    def fixup(ci, mn, rows_v):
        @pl.when(mn == PAD_IDX)
        def _():
            z = jnp.zeros((L,), jnp.float32)

            def fix_body(g, c):
                v = idx_v[pl.ds(ci * CHUNK + g * L, L)]
                for r in range(L):
                    @pl.when(v[r] == PAD_IDX)
                    def _zero_row(row=g * L + r):
                        for h in range(DIM // L):
                            rows_v[row, pl.ds(h * L, L)] = z

                return c

            lax.fori_loop(0, GROUPS, fix_body, 0)

    bufs = ((rows0, sg0, ss0), (rows1, sg1, ss1))

    def do_chunk(ci, rows_v, sg, ss):
        @pl.when(ci >= 2)
        def _():
            pltpu.make_async_copy(
                rows_v, out_hbm.at[pl.ds(wbase, CHUNK)], ss
            ).wait()

        gather = pltpu.async_copy(
            table_hbm.at[idx_v.at[pl.ds(ci * CHUNK, CHUNK)]], rows_v, sg
        )
        mn = pad_scan(ci)
        gather.wait()
        fixup(ci, mn, rows_v)
        pltpu.async_copy(
            rows_v, out_hbm.at[pl.ds(wbase + ci * CHUNK, CHUNK)], ss
        )

    def pair_body(k, carry):
        for b in range(2):
            rows_v, sg, ss = bufs[b]
            do_chunk(2 * k + b, rows_v, sg, ss)
        return carry

    lax.fori_loop(0, NCHUNK // 2, pair_body, 0)

    for b in range(2):
        rows_v, _, ss = bufs[b]
        pltpu.make_async_copy(
            rows_v, out_hbm.at[pl.ds(wbase, CHUNK)], ss
        ).wait()


def kernel(idx, embeddings, padding_mult):
    tail_lin = embeddings[NFULL * RBLK:].reshape(-1)
    lin_flat = _transpose_table(embeddings.T, tail_lin)
    lin_table = lin_flat.reshape(NUM_EMB, DIM)
    idx_flat = idx.reshape(-1)
    mesh = plsc.VectorSubcoreMesh(core_axis_name="c", subcore_axis_name="s")
    out = pl.kernel(
        _body,
        out_type=jax.ShapeDtypeStruct((TOTAL, DIM), jnp.float32),
        mesh=mesh,
        compiler_params=pltpu.CompilerParams(use_tc_tiling_on_sc=False),
        scratch_types=[
            pltpu.VMEM((ROWS_PER_W,), jnp.int32),
            pltpu.VMEM((CHUNK, DIM), jnp.float32),
            pltpu.VMEM((CHUNK, DIM), jnp.float32),
            pltpu.SemaphoreType.DMA,
            pltpu.SemaphoreType.DMA,
            pltpu.SemaphoreType.DMA,
            pltpu.SemaphoreType.DMA,
        ],
    )(idx_flat, lin_table)
    return out.reshape(idx.shape + (DIM,))


# speedup vs baseline: 2.2297x; 1.5278x over previous
import jax
import jax.numpy as jnp
from jax import lax
from jax.experimental import pallas as pl
from jax.experimental.pallas import tpu as pltpu
from jax.experimental.pallas import tpu_sc as plsc

NUM_EMB = 1000000
DIM = 32
PAD_IDX = 0
TOTAL = 4096 * 200
NC, NS, L = 2, 16, 16
NW = NC * NS

RBLK = 128
NFULL = NUM_EMB // RBLK
TAIL = NUM_EMB - NFULL * RBLK


def _tr_body(tbl_t, tail_lin, lin_hbm, t0, t1, l0, l1, sin0, sin1, so0, so1):
    wid = lax.axis_index("s") * NC + lax.axis_index("c")
    iota = lax.broadcasted_iota(jnp.int32, (L,), 0)

    def transpose_block(tile_v, lin_v, nrows):
        nsub = (DIM // L) * (nrows // L)

        def sub_body(t, carry):
            d_base = (t >> 3) * L
            colv = (t & 7) * L + iota
            for k in range(L):
                rowv = d_base + ((iota + k) & (L - 1))
                v = plsc.load_gather(tile_v, [rowv, colv])
                plsc.store_scatter(lin_v, [colv * DIM + rowv], v)
            return carry

        lax.fori_loop(0, nsub, sub_body, 0)

    bufs = ((t0, l0, sin0, so0), (t1, l1, sin1, so1))
    nsteps = 2 * (((NFULL - 1) // NW + 1 + 1) // 2)

    def blk(i):
        return (wid + i * NW) * RBLK

    def active(i):
        return blk(i) < NFULL * RBLK

    def start_in(i, parity):
        tile_v, _, sin, _ = bufs[parity]

        @pl.when(active(i))
        def _():
            pltpu.async_copy(tbl_t.at[:, pl.ds(blk(i), RBLK)], tile_v, sin)

    def step(i, parity, tile_v, lin_v, sin, so):
        start_in(i + 1, 1 - parity)

        @pl.when((i >= 2) & active(i - 2))
        def _():
            pltpu.make_async_copy(
                lin_v, lin_hbm.at[pl.ds(0, RBLK * DIM)], so
            ).wait()

        @pl.when(active(i))
        def _():
            pltpu.make_async_copy(
                tbl_t.at[:, pl.ds(blk(i), RBLK)], tile_v, sin
            ).wait()
            transpose_block(tile_v, lin_v, RBLK)
            pltpu.async_copy(
                lin_v, lin_hbm.at[pl.ds(blk(i) * DIM, RBLK * DIM)], so
            )

    start_in(0, 0)

    def pair_body(k, carry):
        for b in range(2):
            tile_v, lin_v, sin, so = bufs[b]
            step(2 * k + b, b, tile_v, lin_v, sin, so)
        return carry

    lax.fori_loop(0, nsteps // 2, pair_body, 0)
    for b in range(2):
        tile_v, lin_v, sin, so = bufs[b]
        last_i = nsteps - 2 + b
        @pl.when(active(last_i))
        def _():
            pltpu.make_async_copy(
                lin_v, lin_hbm.at[pl.ds(0, RBLK * DIM)], so
            ).wait()

    @pl.when(wid == 0)
    def _():
        pltpu.sync_copy(tail_lin, l0.at[pl.ds(0, TAIL * DIM)])
        pltpu.sync_copy(
            l0.at[pl.ds(0, TAIL * DIM)],
            lin_hbm.at[pl.ds(NFULL * RBLK * DIM, TAIL * DIM)],
        )


def _transpose_table(embeddings_t, tail_lin):
    mesh = plsc.VectorSubcoreMesh(core_axis_name="c", subcore_axis_name="s")
    return pl.kernel(
        _tr_body,
        out_type=jax.ShapeDtypeStruct((NUM_EMB * DIM,), jnp.float32),
        mesh=mesh,
        compiler_params=pltpu.CompilerParams(
            use_tc_tiling_on_sc=True, needs_layout_passes=False),
        scratch_types=[
            pltpu.VMEM((DIM, RBLK), jnp.float32),
            pltpu.VMEM((DIM, RBLK), jnp.float32),
            pltpu.VMEM((RBLK * DIM,), jnp.float32),
            pltpu.VMEM((RBLK * DIM,), jnp.float32),
            pltpu.SemaphoreType.DMA,
            pltpu.SemaphoreType.DMA,
            pltpu.SemaphoreType.DMA,
            pltpu.SemaphoreType.DMA,
        ],
    )(embeddings_t, tail_lin)


ROWS_PER_W = TOTAL // NW
G = 128
NGRP_W = ROWS_PER_W // G
GVREG = G // L


def _body(idx_hbm, table_hbm, out_hbm, idx_v, r0b, r1b, o0, o1,
          sg0, sg1, ss0, ss1):
    wid = lax.axis_index("s") * NC + lax.axis_index("c")
    wbase = wid * ROWS_PER_W
    pltpu.sync_copy(idx_hbm.at[pl.ds(wbase, ROWS_PER_W)], idx_v)
    iota = lax.broadcasted_iota(jnp.int32, (L,), 0)

    def grp(i):
        return wid * NGRP_W + i

    bufs = ((r0b, o0, sg0, ss0), (r1b, o1, sg1, ss1))

    def start_gather(i, parity):
        rows_v, _, sg, _ = bufs[parity]

        @pl.when(i < NGRP_W)
        def _():
            pltpu.async_copy(
                table_hbm.at[idx_v.at[pl.ds(i * G, G)]], rows_v, sg
            )

    def pad_scan(i):
        def scan_body(g, acc):
            return jnp.minimum(acc, idx_v[pl.ds(i * G + g * L, L)])

        acc = lax.fori_loop(
            0, GVREG, scan_body, jnp.full((L,), NUM_EMB, jnp.int32)
        )
        mn = acc[0]
        for j in range(1, L):
            mn = jnp.minimum(mn, acc[j])
        return mn

    def fixup(i, mn, rows_v):
        @pl.when(mn == PAD_IDX)
        def _():
            z = jnp.zeros((L,), jnp.float32)

            def fix_body(g, c):
                v = idx_v[pl.ds(i * G + g * L, L)]
                for r in range(L):
                    @pl.when(v[r] == PAD_IDX)
                    def _zero_row(row=g * L + r):
                        for h in range(DIM // L):
                            rows_v[row, pl.ds(h * L, L)] = z

                return c

            lax.fori_loop(0, GVREG, fix_body, 0)

    def transpose_group(rows_v, out_t):
        def sub_body(t, carry):
            d_base = (t >> 3) * L
            bv = (t & 7) * L + iota
            for k in range(L):
                dv = d_base + ((iota + k) & (L - 1))
                v = plsc.load_gather(rows_v, [bv, dv])
                plsc.store_scatter(out_t, [dv * G + bv], v)
            return carry

        lax.fori_loop(0, (DIM // L) * (G // L), sub_body, 0)

    def out_slices(i):
        g = grp(i)
        h = g >> 5
        bb = g & 31
        return [(db, h, bb) for db in range(4)]

    def step(i, parity, rows_v, out_t, sg, ss):
        start_gather(i + 1, 1 - parity)

        @pl.when(i >= 2)
        def _():
            for db, h, bb in out_slices(i - 2):
                pltpu.make_async_copy(
                    out_t.at[pl.ds(db * 1024, 1024)],
                    out_hbm.at[h, db, bb], ss
                ).wait()

        pltpu.make_async_copy(
            table_hbm.at[idx_v.at[pl.ds(i * G, G)]], rows_v, sg
        ).wait()
        mn = pad_scan(i)
        fixup(i, mn, rows_v)
        transpose_group(rows_v, out_t)
        for db, h, bb in out_slices(i):
            pltpu.async_copy(
                out_t.at[pl.ds(db * 1024, 1024)], out_hbm.at[h, db, bb], ss
            )

    start_gather(0, 0)

    def pair_body(k, carry):
        for b in range(2):
            rows_v, out_t, sg, ss = bufs[b]
            step(2 * k + b, b, rows_v, out_t, sg, ss)
        return carry

    lax.fori_loop(0, NGRP_W // 2, pair_body, 0)

    for b in range(2):
        rows_v, out_t, _, ss = bufs[b]
        for db, h, bb in out_slices(NGRP_W - 2 + b):
            pltpu.make_async_copy(
                out_t.at[pl.ds(db * 1024, 1024)], out_hbm.at[h, db, bb], ss
            ).wait()


def kernel(idx, embeddings, padding_mult):
    tail_lin = embeddings[NFULL * RBLK:].reshape(-1)
    lin_flat = _transpose_table(embeddings.T, tail_lin)
    lin_table = lin_flat.reshape(NUM_EMB, DIM)
    idx_flat = idx.T.reshape(-1)
    mesh = plsc.VectorSubcoreMesh(core_axis_name="c", subcore_axis_name="s")
    out5d = pl.kernel(
        _body,
        out_type=jax.ShapeDtypeStruct((200, 4, 32, 1024), jnp.float32),
        mesh=mesh,
        compiler_params=pltpu.CompilerParams(
            use_tc_tiling_on_sc=False, needs_layout_passes=False),
        scratch_types=[
            pltpu.VMEM((ROWS_PER_W,), jnp.int32),
            pltpu.VMEM((G, DIM), jnp.float32),
            pltpu.VMEM((G, DIM), jnp.float32),
            pltpu.VMEM((4 * 1024,), jnp.float32),
            pltpu.VMEM((4 * 1024,), jnp.float32),
            pltpu.SemaphoreType.DMA,
            pltpu.SemaphoreType.DMA,
            pltpu.SemaphoreType.DMA,
            pltpu.SemaphoreType.DMA,
        ],
    )(idx_flat, lin_table)
    out = (out5d.reshape(200, 4, 32, 8, 128)
           .transpose(2, 4, 0, 1, 3)
           .reshape(4096, 200, 32))
    return out


# speedup vs baseline: 2.3375x; 1.0483x over previous
import jax
import jax.numpy as jnp
from jax import lax
from jax.experimental import pallas as pl
from jax.experimental.pallas import tpu as pltpu
from jax.experimental.pallas import tpu_sc as plsc

NUM_EMB = 1000000
DIM = 32
PAD_IDX = 0
TOTAL = 4096 * 200
NC, NS, L = 2, 16, 16
NW = NC * NS

RBLK = 128
NFULL = NUM_EMB // RBLK
TAIL = NUM_EMB - NFULL * RBLK


def _tr_body(tbl_t, tail_lin, lin_hbm, t0, t1, l0, l1, sin0, sin1, so0, so1):
    wid = lax.axis_index("s") * NC + lax.axis_index("c")
    iota = lax.broadcasted_iota(jnp.int32, (L,), 0)

    def transpose_block(tile_v, lin_v, nrows):
        def sub_body(c, carry):
            colv = c * L + iota
            for d_base in range(0, DIM, L):
                for k in range(L):
                    rowv = d_base + ((iota + k) & (L - 1))
                    v = plsc.load_gather(tile_v, [rowv, colv])
                    plsc.store_scatter(lin_v, [iota * DIM + rowv + c * (L * DIM)], v)
            return carry

        lax.fori_loop(0, nrows // L, sub_body, 0)

    bufs = ((t0, l0, sin0, so0), (t1, l1, sin1, so1))
    nsteps = 2 * (((NFULL - 1) // NW + 1 + 1) // 2)

    def blk(i):
        return (wid + i * NW) * RBLK

    def active(i):
        return blk(i) < NFULL * RBLK

    def start_in(i, parity):
        tile_v, _, sin, _ = bufs[parity]

        @pl.when(active(i))
        def _():
            pltpu.async_copy(tbl_t.at[:, pl.ds(blk(i), RBLK)], tile_v, sin)

    def step(i, parity, tile_v, lin_v, sin, so):
        start_in(i + 1, 1 - parity)

        @pl.when((i >= 2) & active(i - 2))
        def _():
            pltpu.make_async_copy(
                lin_v, lin_hbm.at[pl.ds(0, RBLK * DIM)], so
            ).wait()

        @pl.when(active(i))
        def _():
            pltpu.make_async_copy(
                tbl_t.at[:, pl.ds(blk(i), RBLK)], tile_v, sin
            ).wait()
            transpose_block(tile_v, lin_v, RBLK)
            pltpu.async_copy(
                lin_v, lin_hbm.at[pl.ds(blk(i) * DIM, RBLK * DIM)], so
            )

    start_in(0, 0)

    def pair_body(k, carry):
        for b in range(2):
            tile_v, lin_v, sin, so = bufs[b]
            step(2 * k + b, b, tile_v, lin_v, sin, so)
        return carry

    lax.fori_loop(0, nsteps // 2, pair_body, 0)
    for b in range(2):
        tile_v, lin_v, sin, so = bufs[b]
        last_i = nsteps - 2 + b
        @pl.when(active(last_i))
        def _():
            pltpu.make_async_copy(
                lin_v, lin_hbm.at[pl.ds(0, RBLK * DIM)], so
            ).wait()

    @pl.when(wid == 0)
    def _():
        pltpu.sync_copy(tail_lin, l0.at[pl.ds(0, TAIL * DIM)])
        pltpu.sync_copy(
            l0.at[pl.ds(0, TAIL * DIM)],
            lin_hbm.at[pl.ds(NFULL * RBLK * DIM, TAIL * DIM)],
        )


def _transpose_table(embeddings_t, tail_lin):
    mesh = plsc.VectorSubcoreMesh(core_axis_name="c", subcore_axis_name="s")
    return pl.kernel(
        _tr_body,
        out_type=jax.ShapeDtypeStruct((NUM_EMB * DIM,), jnp.float32),
        mesh=mesh,
        compiler_params=pltpu.CompilerParams(
            use_tc_tiling_on_sc=True, needs_layout_passes=False),
        scratch_types=[
            pltpu.VMEM((DIM, RBLK), jnp.float32),
            pltpu.VMEM((DIM, RBLK), jnp.float32),
            pltpu.VMEM((RBLK * DIM,), jnp.float32),
            pltpu.VMEM((RBLK * DIM,), jnp.float32),
            pltpu.SemaphoreType.DMA,
            pltpu.SemaphoreType.DMA,
            pltpu.SemaphoreType.DMA,
            pltpu.SemaphoreType.DMA,
        ],
    )(embeddings_t, tail_lin)


ROWS_PER_W = TOTAL // NW
G = 128
NGRP_W = ROWS_PER_W // G
GVREG = G // L


def _body(idx_hbm, table_hbm, out_hbm, idx_v, r0b, r1b, o0, o1,
          sg0, sg1, ss0, ss1):
    wid = lax.axis_index("s") * NC + lax.axis_index("c")
    wbase = wid * ROWS_PER_W
    pltpu.sync_copy(idx_hbm.at[pl.ds(wbase, ROWS_PER_W)], idx_v)
    iota = lax.broadcasted_iota(jnp.int32, (L,), 0)

    def grp(i):
        return wid * NGRP_W + i

    bufs = ((r0b, o0, sg0, ss0), (r1b, o1, sg1, ss1))

    def start_gather(i, parity):
        rows_v, _, sg, _ = bufs[parity]

        @pl.when(i < NGRP_W)
        def _():
            pltpu.async_copy(
                table_hbm.at[idx_v.at[pl.ds(i * G, G)]], rows_v, sg
            )

    def pad_scan(i):
        def scan_body(g, acc):
            return jnp.minimum(acc, idx_v[pl.ds(i * G + g * L, L)])

        acc = lax.fori_loop(
            0, GVREG, scan_body, jnp.full((L,), NUM_EMB, jnp.int32)
        )
        mn = acc[0]
        for j in range(1, L):
            mn = jnp.minimum(mn, acc[j])
        return mn

    def fixup(i, mn, rows_v):
        @pl.when(mn == PAD_IDX)
        def _():
            z = jnp.zeros((L,), jnp.float32)

            def fix_body(g, c):
                v = idx_v[pl.ds(i * G + g * L, L)]
                for r in range(L):
                    @pl.when(v[r] == PAD_IDX)
                    def _zero_row(row=g * L + r):
                        for h in range(DIM // L):
                            rows_v[row, pl.ds(h * L, L)] = z

                return c

            lax.fori_loop(0, GVREG, fix_body, 0)

    def transpose_group(rows_v, out_t):
        def sub_body(c, carry):
            bv = c * L + iota
            for d_base in range(0, DIM, L):
                for k in range(L):
                    dv = d_base + ((iota + k) & (L - 1))
                    v = plsc.load_gather(rows_v, [bv, dv])
                    plsc.store_scatter(out_t, [dv * G + iota + c * L], v)
            return carry

        lax.fori_loop(0, G // L, sub_body, 0)

    def out_slices(i):
        g = grp(i)
        h = g >> 5
        bb = g & 31
        return [(db, h, bb) for db in range(4)]

    def step(i, parity, rows_v, out_t, sg, ss):
        start_gather(i + 1, 1 - parity)

        @pl.when(i >= 2)
        def _():
            for db, h, bb in out_slices(i - 2):
                pltpu.make_async_copy(
                    out_t.at[pl.ds(db * 1024, 1024)],
                    out_hbm.at[h, db, bb], ss
                ).wait()

        pltpu.make_async_copy(
            table_hbm.at[idx_v.at[pl.ds(i * G, G)]], rows_v, sg
        ).wait()
        mn = pad_scan(i)
        fixup(i, mn, rows_v)
        transpose_group(rows_v, out_t)
        for db, h, bb in out_slices(i):
            pltpu.async_copy(
                out_t.at[pl.ds(db * 1024, 1024)], out_hbm.at[h, db, bb], ss
            )

    start_gather(0, 0)

    def pair_body(k, carry):
        for b in range(2):
            rows_v, out_t, sg, ss = bufs[b]
            step(2 * k + b, b, rows_v, out_t, sg, ss)
        return carry

    lax.fori_loop(0, NGRP_W // 2, pair_body, 0)

    for b in range(2):
        rows_v, out_t, _, ss = bufs[b]
        for db, h, bb in out_slices(NGRP_W - 2 + b):
            pltpu.make_async_copy(
                out_t.at[pl.ds(db * 1024, 1024)], out_hbm.at[h, db, bb], ss
            ).wait()


def kernel(idx, embeddings, padding_mult):
    tail_lin = embeddings[NFULL * RBLK:].reshape(-1)
    lin_flat = _transpose_table(embeddings.T, tail_lin)
    lin_table = lin_flat.reshape(NUM_EMB, DIM)
    idx_flat = idx.T.reshape(-1)
    mesh = plsc.VectorSubcoreMesh(core_axis_name="c", subcore_axis_name="s")
    out5d = pl.kernel(
        _body,
        out_type=jax.ShapeDtypeStruct((200, 4, 32, 1024), jnp.float32),
        mesh=mesh,
        compiler_params=pltpu.CompilerParams(
            use_tc_tiling_on_sc=False, needs_layout_passes=False),
        scratch_types=[
            pltpu.VMEM((ROWS_PER_W,), jnp.int32),
            pltpu.VMEM((G, DIM), jnp.float32),
            pltpu.VMEM((G, DIM), jnp.float32),
            pltpu.VMEM((4 * 1024,), jnp.float32),
            pltpu.VMEM((4 * 1024,), jnp.float32),
            pltpu.SemaphoreType.DMA,
            pltpu.SemaphoreType.DMA,
            pltpu.SemaphoreType.DMA,
            pltpu.SemaphoreType.DMA,
        ],
    )(idx_flat, lin_table)
    out = (out5d.reshape(200, 4, 32, 8, 128)
           .transpose(2, 4, 0, 1, 3)
           .reshape(4096, 200, 32))
    return out


# speedup vs baseline: 2.9856x; 1.2773x over previous
import jax
import jax.numpy as jnp
from jax import lax
from jax.experimental import pallas as pl
from jax.experimental.pallas import tpu as pltpu
from jax.experimental.pallas import tpu_sc as plsc

NUM_EMB = 1000000
DIM = 32
PAD_IDX = 0
TOTAL = 4096 * 200
NC, NS, L = 2, 16, 16
NW = NC * NS

RBLK = 128
NFULL = NUM_EMB // RBLK
TAIL = NUM_EMB - NFULL * RBLK


def _tr_body(tbl_t, tail_lin, lin_hbm, t0, t1, l0, l1, sin0, sin1, so0, so1):
    wid = lax.axis_index("s") * NC + lax.axis_index("c")
    iota = lax.broadcasted_iota(jnp.int32, (L,), 0)

    def transpose_block(tile_v, lin_v, nrows):
        def sub_body(c, carry):
            colv = c * L + iota
            for d_base in range(0, DIM, L):
                skews = [d_base + ((iota + k) & (L - 1)) for k in range(L)]
                vs = [plsc.load_gather(tile_v, [rowv, colv]) for rowv in skews]
                for rowv, v in zip(skews, vs):
                    plsc.store_scatter(
                        lin_v, [iota * DIM + rowv + c * (L * DIM)], v
                    )
            return carry

        lax.fori_loop(0, nrows // L, sub_body, 0)

    bufs = ((t0, l0, sin0, so0), (t1, l1, sin1, so1))
    nsteps = 2 * (((NFULL - 1) // NW + 1 + 1) // 2)

    def blk(i):
        return (wid + i * NW) * RBLK

    def active(i):
        return blk(i) < NFULL * RBLK

    def start_in(i, parity):
        tile_v, _, sin, _ = bufs[parity]

        @pl.when(active(i))
        def _():
            pltpu.async_copy(tbl_t.at[:, pl.ds(blk(i), RBLK)], tile_v, sin)

    def step(i, parity, tile_v, lin_v, sin, so):
        start_in(i + 1, 1 - parity)

        @pl.when((i >= 2) & active(i - 2))
        def _():
            pltpu.make_async_copy(
                lin_v, lin_hbm.at[pl.ds(0, RBLK * DIM)], so
            ).wait()

        @pl.when(active(i))
        def _():
            pltpu.make_async_copy(
                tbl_t.at[:, pl.ds(blk(i), RBLK)], tile_v, sin
            ).wait()
            transpose_block(tile_v, lin_v, RBLK)
            pltpu.async_copy(
                lin_v, lin_hbm.at[pl.ds(blk(i) * DIM, RBLK * DIM)], so
            )

    start_in(0, 0)

    def pair_body(k, carry):
        for b in range(2):
            tile_v, lin_v, sin, so = bufs[b]
            step(2 * k + b, b, tile_v, lin_v, sin, so)
        return carry

    lax.fori_loop(0, nsteps // 2, pair_body, 0)
    for b in range(2):
        tile_v, lin_v, sin, so = bufs[b]
        last_i = nsteps - 2 + b
        @pl.when(active(last_i))
        def _():
            pltpu.make_async_copy(
                lin_v, lin_hbm.at[pl.ds(0, RBLK * DIM)], so
            ).wait()

    @pl.when(wid == 0)
    def _():
        pltpu.sync_copy(tail_lin, l0.at[pl.ds(0, TAIL * DIM)])
        pltpu.sync_copy(
            l0.at[pl.ds(0, TAIL * DIM)],
            lin_hbm.at[pl.ds(NFULL * RBLK * DIM, TAIL * DIM)],
        )


def _transpose_table(embeddings_t, tail_lin):
    mesh = plsc.VectorSubcoreMesh(core_axis_name="c", subcore_axis_name="s")
    return pl.kernel(
        _tr_body,
        out_type=jax.ShapeDtypeStruct((NUM_EMB * DIM,), jnp.float32),
        mesh=mesh,
        compiler_params=pltpu.CompilerParams(
            use_tc_tiling_on_sc=True, needs_layout_passes=False),
        scratch_types=[
            pltpu.VMEM((DIM, RBLK), jnp.float32),
            pltpu.VMEM((DIM, RBLK), jnp.float32),
            pltpu.VMEM((RBLK * DIM,), jnp.float32),
            pltpu.VMEM((RBLK * DIM,), jnp.float32),
            pltpu.SemaphoreType.DMA,
            pltpu.SemaphoreType.DMA,
            pltpu.SemaphoreType.DMA,
            pltpu.SemaphoreType.DMA,
        ],
    )(embeddings_t, tail_lin)


ROWS_PER_W = TOTAL // NW
G = 128
NGRP_W = ROWS_PER_W // G
GVREG = G // L


def _body(idx_hbm, table_hbm, out_hbm, idx_v, r0b, r1b, o0, o1,
          sg0, sg1, ss0, ss1):
    wid = lax.axis_index("s") * NC + lax.axis_index("c")
    wbase = wid * ROWS_PER_W
    pltpu.sync_copy(idx_hbm.at[pl.ds(wbase, ROWS_PER_W)], idx_v)
    iota = lax.broadcasted_iota(jnp.int32, (L,), 0)

    def grp(i):
        return wid * NGRP_W + i

    bufs = ((r0b, o0, sg0, ss0), (r1b, o1, sg1, ss1))

    def start_gather(i, parity):
        rows_v, _, sg, _ = bufs[parity]

        @pl.when(i < NGRP_W)
        def _():
            pltpu.async_copy(
                table_hbm.at[idx_v.at[pl.ds(i * G, G)]], rows_v, sg
            )

    def pad_scan(i):
        def scan_body(g, acc):
            return jnp.minimum(acc, idx_v[pl.ds(i * G + g * L, L)])

        acc = lax.fori_loop(
            0, GVREG, scan_body, jnp.full((L,), NUM_EMB, jnp.int32)
        )
        mn = acc[0]
        for j in range(1, L):
            mn = jnp.minimum(mn, acc[j])
        return mn

    def fixup(i, mn, rows_v):
        @pl.when(mn == PAD_IDX)
        def _():
            z = jnp.zeros((L,), jnp.float32)

            def fix_body(g, c):
                v = idx_v[pl.ds(i * G + g * L, L)]
                for r in range(L):
                    @pl.when(v[r] == PAD_IDX)
                    def _zero_row(row=g * L + r):
                        for h in range(DIM // L):
                            rows_v[row, pl.ds(h * L, L)] = z

                return c

            lax.fori_loop(0, GVREG, fix_body, 0)

    def transpose_group(rows_v, out_t):
        def sub_body(c, carry):
            bv = c * L + iota
            for d_base in range(0, DIM, L):
                skews = [d_base + ((iota + k) & (L - 1)) for k in range(L)]
                vs = [plsc.load_gather(rows_v, [bv, dv]) for dv in skews]
                for dv, v in zip(skews, vs):
                    plsc.store_scatter(out_t, [dv * G + iota + c * L], v)
            return carry

        lax.fori_loop(0, G // L, sub_body, 0)

    def out_slices(i):
        g = grp(i)
        h = g >> 5
        bb = g & 31
        return [(db, h, bb) for db in range(4)]

    def step(i, parity, rows_v, out_t, sg, ss):
        start_gather(i + 1, 1 - parity)

        @pl.when(i >= 2)
        def _():
            for db, h, bb in out_slices(i - 2):
                pltpu.make_async_copy(
                    out_t.at[pl.ds(db * 1024, 1024)],
                    out_hbm.at[h, db, bb], ss
                ).wait()

        pltpu.make_async_copy(
            table_hbm.at[idx_v.at[pl.ds(i * G, G)]], rows_v, sg
        ).wait()
        mn = pad_scan(i)
        fixup(i, mn, rows_v)
        transpose_group(rows_v, out_t)
        for db, h, bb in out_slices(i):
            pltpu.async_copy(
                out_t.at[pl.ds(db * 1024, 1024)], out_hbm.at[h, db, bb], ss
            )

    start_gather(0, 0)

    def pair_body(k, carry):
        for b in range(2):
            rows_v, out_t, sg, ss = bufs[b]
            step(2 * k + b, b, rows_v, out_t, sg, ss)
        return carry

    lax.fori_loop(0, NGRP_W // 2, pair_body, 0)

    for b in range(2):
        rows_v, out_t, _, ss = bufs[b]
        for db, h, bb in out_slices(NGRP_W - 2 + b):
            pltpu.make_async_copy(
                out_t.at[pl.ds(db * 1024, 1024)], out_hbm.at[h, db, bb], ss
            ).wait()


def kernel(idx, embeddings, padding_mult):
    tail_lin = embeddings[NFULL * RBLK:].reshape(-1)
    lin_flat = _transpose_table(embeddings.T, tail_lin)
    lin_table = lin_flat.reshape(NUM_EMB, DIM)
    idx_flat = idx.T.reshape(-1)
    mesh = plsc.VectorSubcoreMesh(core_axis_name="c", subcore_axis_name="s")
    out5d = pl.kernel(
        _body,
        out_type=jax.ShapeDtypeStruct((200, 4, 32, 1024), jnp.float32),
        mesh=mesh,
        compiler_params=pltpu.CompilerParams(
            use_tc_tiling_on_sc=False, needs_layout_passes=False),
        scratch_types=[
            pltpu.VMEM((ROWS_PER_W,), jnp.int32),
            pltpu.VMEM((G, DIM), jnp.float32),
            pltpu.VMEM((G, DIM), jnp.float32),
            pltpu.VMEM((4 * 1024,), jnp.float32),
            pltpu.VMEM((4 * 1024,), jnp.float32),
            pltpu.SemaphoreType.DMA,
            pltpu.SemaphoreType.DMA,
            pltpu.SemaphoreType.DMA,
            pltpu.SemaphoreType.DMA,
        ],
    )(idx_flat, lin_table)
    out = (out5d.reshape(200, 4, 32, 8, 128)
           .transpose(2, 4, 0, 1, 3)
           .reshape(4096, 200, 32))
    return out


# speedup vs baseline: 3.2193x; 1.0783x over previous
import jax
import jax.numpy as jnp
from jax import lax
from jax.experimental import pallas as pl
from jax.experimental.pallas import tpu as pltpu
from jax.experimental.pallas import tpu_sc as plsc

NUM_EMB = 1000000
DIM = 32
PAD_IDX = 0
TOTAL = 4096 * 200
NC, NS, L = 2, 16, 16
NW = NC * NS

RBLK = 128
NFULL = NUM_EMB // RBLK
TAIL = NUM_EMB - NFULL * RBLK


def _tr_body(tbl_t, tail_lin, lin_hbm, t0, t1, l0, l1, sin0, sin1, so0, so1):
    wid = lax.axis_index("s") * NC + lax.axis_index("c")
    iota = lax.broadcasted_iota(jnp.int32, (L,), 0)

    def transpose_block(tile_v, lin_v, nrows):
        def sub_body(ch, carry):
            for u in range(2):
                c = ch * 2 + u
                colv = c * L + iota
                for d_base in range(0, DIM, L):
                    skews = [d_base + ((iota + k) & (L - 1)) for k in range(L)]
                    vs = [
                        plsc.load_gather(tile_v, [rowv, colv])
                        for rowv in skews
                    ]
                    for rowv, v in zip(skews, vs):
                        plsc.store_scatter(
                            lin_v, [iota * DIM + rowv + c * (L * DIM)], v
                        )
            return carry

        lax.fori_loop(0, nrows // L // 2, sub_body, 0)

    bufs = ((t0, l0, sin0, so0), (t1, l1, sin1, so1))
    nsteps = 2 * (((NFULL - 1) // NW + 1 + 1) // 2)

    def blk(i):
        return (wid + i * NW) * RBLK

    def active(i):
        return blk(i) < NFULL * RBLK

    def start_in(i, parity):
        tile_v, _, sin, _ = bufs[parity]

        @pl.when(active(i))
        def _():
            pltpu.async_copy(tbl_t.at[:, pl.ds(blk(i), RBLK)], tile_v, sin)

    def step(i, parity, tile_v, lin_v, sin, so):
        start_in(i + 1, 1 - parity)

        @pl.when((i >= 2) & active(i - 2))
        def _():
            pltpu.make_async_copy(
                lin_v, lin_hbm.at[pl.ds(0, RBLK * DIM)], so
            ).wait()

        @pl.when(active(i))
        def _():
            pltpu.make_async_copy(
                tbl_t.at[:, pl.ds(blk(i), RBLK)], tile_v, sin
            ).wait()
            transpose_block(tile_v, lin_v, RBLK)
            pltpu.async_copy(
                lin_v, lin_hbm.at[pl.ds(blk(i) * DIM, RBLK * DIM)], so
            )

    start_in(0, 0)

    def pair_body(k, carry):
        for b in range(2):
            tile_v, lin_v, sin, so = bufs[b]
            step(2 * k + b, b, tile_v, lin_v, sin, so)
        return carry

    lax.fori_loop(0, nsteps // 2, pair_body, 0)
    for b in range(2):
        tile_v, lin_v, sin, so = bufs[b]
        last_i = nsteps - 2 + b
        @pl.when(active(last_i))
        def _():
            pltpu.make_async_copy(
                lin_v, lin_hbm.at[pl.ds(0, RBLK * DIM)], so
            ).wait()

    @pl.when(wid == 0)
    def _():
        pltpu.sync_copy(tail_lin, l0.at[pl.ds(0, TAIL * DIM)])
        pltpu.sync_copy(
            l0.at[pl.ds(0, TAIL * DIM)],
            lin_hbm.at[pl.ds(NFULL * RBLK * DIM, TAIL * DIM)],
        )


def _transpose_table(embeddings_t, tail_lin):
    mesh = plsc.VectorSubcoreMesh(core_axis_name="c", subcore_axis_name="s")
    return pl.kernel(
        _tr_body,
        out_type=jax.ShapeDtypeStruct((NUM_EMB * DIM,), jnp.float32),
        mesh=mesh,
        compiler_params=pltpu.CompilerParams(
            use_tc_tiling_on_sc=True, needs_layout_passes=False),
        scratch_types=[
            pltpu.VMEM((DIM, RBLK), jnp.float32),
            pltpu.VMEM((DIM, RBLK), jnp.float32),
            pltpu.VMEM((RBLK * DIM,), jnp.float32),
            pltpu.VMEM((RBLK * DIM,), jnp.float32),
            pltpu.SemaphoreType.DMA,
            pltpu.SemaphoreType.DMA,
            pltpu.SemaphoreType.DMA,
            pltpu.SemaphoreType.DMA,
        ],
    )(embeddings_t, tail_lin)


ROWS_PER_W = TOTAL // NW
G = 128
NGRP_W = ROWS_PER_W // G
GVREG = G // L


def _body(idx_hbm, table_hbm, out_hbm, idx_v, r0b, r1b, o0, o1,
          sg0, sg1, ss0, ss1):
    wid = lax.axis_index("s") * NC + lax.axis_index("c")
    wbase = wid * ROWS_PER_W
    pltpu.sync_copy(idx_hbm.at[pl.ds(wbase, ROWS_PER_W)], idx_v)
    iota = lax.broadcasted_iota(jnp.int32, (L,), 0)

    def grp(i):
        return wid * NGRP_W + i

    bufs = ((r0b, o0, sg0, ss0), (r1b, o1, sg1, ss1))

    def start_gather(i, parity):
        rows_v, _, sg, _ = bufs[parity]

        @pl.when(i < NGRP_W)
        def _():
            pltpu.async_copy(
                table_hbm.at[idx_v.at[pl.ds(i * G, G)]], rows_v, sg
            )

    def pad_scan(i):
        def scan_body(g, acc):
            return jnp.minimum(acc, idx_v[pl.ds(i * G + g * L, L)])

        acc = lax.fori_loop(
            0, GVREG, scan_body, jnp.full((L,), NUM_EMB, jnp.int32)
        )
        mn = acc[0]
        for j in range(1, L):
            mn = jnp.minimum(mn, acc[j])
        return mn

    def fixup(i, mn, rows_v):
        @pl.when(mn == PAD_IDX)
        def _():
            z = jnp.zeros((L,), jnp.float32)

            def fix_body(g, c):
                v = idx_v[pl.ds(i * G + g * L, L)]
                for r in range(L):
                    @pl.when(v[r] == PAD_IDX)
                    def _zero_row(row=g * L + r):
                        for h in range(DIM // L):
                            rows_v[row, pl.ds(h * L, L)] = z

                return c

            lax.fori_loop(0, GVREG, fix_body, 0)

    def transpose_group(rows_v, out_t):
        def sub_body(ch, carry):
            for u in range(2):
                c = ch * 2 + u
                bv = c * L + iota
                for d_base in range(0, DIM, L):
                    skews = [d_base + ((iota + k) & (L - 1)) for k in range(L)]
                    vs = [plsc.load_gather(rows_v, [bv, dv]) for dv in skews]
                    for dv, v in zip(skews, vs):
                        plsc.store_scatter(out_t, [dv * G + iota + c * L], v)
            return carry

        lax.fori_loop(0, G // L // 2, sub_body, 0)

    def out_slices(i):
        g = grp(i)
        h = g >> 5
        bb = g & 31
        return [(db, h, bb) for db in range(4)]

    def step(i, parity, rows_v, out_t, sg, ss):
        start_gather(i + 1, 1 - parity)

        @pl.when(i >= 2)
        def _():
            for db, h, bb in out_slices(i - 2):
                pltpu.make_async_copy(
                    out_t.at[pl.ds(db * 1024, 1024)],
                    out_hbm.at[h, db, bb], ss
                ).wait()

        mn = pad_scan(i)
        pltpu.make_async_copy(
            table_hbm.at[idx_v.at[pl.ds(i * G, G)]], rows_v, sg
        ).wait()
        fixup(i, mn, rows_v)
        transpose_group(rows_v, out_t)
        for db, h, bb in out_slices(i):
            pltpu.async_copy(
                out_t.at[pl.ds(db * 1024, 1024)], out_hbm.at[h, db, bb], ss
            )

    start_gather(0, 0)

    def pair_body(k, carry):
        for b in range(2):
            rows_v, out_t, sg, ss = bufs[b]
            step(2 * k + b, b, rows_v, out_t, sg, ss)
        return carry

    lax.fori_loop(0, NGRP_W // 2, pair_body, 0)

    for b in range(2):
        rows_v, out_t, _, ss = bufs[b]
        for db, h, bb in out_slices(NGRP_W - 2 + b):
            pltpu.make_async_copy(
                out_t.at[pl.ds(db * 1024, 1024)], out_hbm.at[h, db, bb], ss
            ).wait()


def kernel(idx, embeddings, padding_mult):
    tail_lin = embeddings[NFULL * RBLK:].reshape(-1)
    lin_flat = _transpose_table(embeddings.T, tail_lin)
    lin_table = lin_flat.reshape(NUM_EMB, DIM)
    idx_flat = idx.T.reshape(-1)
    mesh = plsc.VectorSubcoreMesh(core_axis_name="c", subcore_axis_name="s")
    out5d = pl.kernel(
        _body,
        out_type=jax.ShapeDtypeStruct((200, 4, 32, 1024), jnp.float32),
        mesh=mesh,
        compiler_params=pltpu.CompilerParams(
            use_tc_tiling_on_sc=False, needs_layout_passes=False),
        scratch_types=[
            pltpu.VMEM((ROWS_PER_W,), jnp.int32),
            pltpu.VMEM((G, DIM), jnp.float32),
            pltpu.VMEM((G, DIM), jnp.float32),
            pltpu.VMEM((4 * 1024,), jnp.float32),
            pltpu.VMEM((4 * 1024,), jnp.float32),
            pltpu.SemaphoreType.DMA,
            pltpu.SemaphoreType.DMA,
            pltpu.SemaphoreType.DMA,
            pltpu.SemaphoreType.DMA,
        ],
    )(idx_flat, lin_table)
    out = (out5d.reshape(200, 4, 32, 8, 128)
           .transpose(2, 4, 0, 1, 3)
           .reshape(4096, 200, 32))
    return out


# speedup vs baseline: 3.5699x; 1.1089x over previous
import jax
import jax.numpy as jnp
from jax import lax
from jax.experimental import pallas as pl
from jax.experimental.pallas import tpu as pltpu
from jax.experimental.pallas import tpu_sc as plsc

NUM_EMB = 1000000
DIM = 32
PAD_IDX = 0
TOTAL = 4096 * 200
NC, NS, L = 2, 16, 16
NW = NC * NS

RBLK = 128
NFULL = NUM_EMB // RBLK
TAIL = NUM_EMB - NFULL * RBLK


def _tr_body(tbl_t, tail_lin, lin_hbm, t0, t1, l0, l1, sin0, sin1, so0, so1):
    wid = lax.axis_index("s") * NC + lax.axis_index("c")
    iota = lax.broadcasted_iota(jnp.int32, (L,), 0)

    def transpose_block(tile_v, lin_v, nrows):
        def sub_body(ch, carry):
            for u in range(2):
                c = ch * 2 + u
                colv = c * L + iota
                for d_base in range(0, DIM, L):
                    skews = [d_base + ((iota + k) & (L - 1)) for k in range(L)]
                    vs = [
                        plsc.load_gather(tile_v, [rowv, colv])
                        for rowv in skews
                    ]
                    for rowv, v in zip(skews, vs):
                        plsc.store_scatter(
                            lin_v, [iota * DIM + rowv + c * (L * DIM)], v
                        )
            return carry

        lax.fori_loop(0, nrows // L // 2, sub_body, 0)

    bufs = ((t0, l0, sin0, so0), (t1, l1, sin1, so1))
    nsteps = 2 * (((NFULL - 1) // NW + 1 + 1) // 2)

    def blk(i):
        return (wid + i * NW) * RBLK

    def active(i):
        return blk(i) < NFULL * RBLK

    def start_in(i, parity):
        tile_v, _, sin, _ = bufs[parity]

        @pl.when(active(i))
        def _():
            pltpu.async_copy(tbl_t.at[:, pl.ds(blk(i), RBLK)], tile_v, sin)

    def step(i, parity, tile_v, lin_v, sin, so):
        start_in(i + 1, 1 - parity)

        @pl.when((i >= 2) & active(i - 2))
        def _():
            pltpu.make_async_copy(
                lin_v, lin_hbm.at[pl.ds(0, RBLK * DIM)], so
            ).wait()

        @pl.when(active(i))
        def _():
            pltpu.make_async_copy(
                tbl_t.at[:, pl.ds(blk(i), RBLK)], tile_v, sin
            ).wait()
            transpose_block(tile_v, lin_v, RBLK)
            pltpu.async_copy(
                lin_v, lin_hbm.at[pl.ds(blk(i) * DIM, RBLK * DIM)], so
            )

    start_in(0, 0)

    def pair_body(k, carry):
        for b in range(2):
            tile_v, lin_v, sin, so = bufs[b]
            step(2 * k + b, b, tile_v, lin_v, sin, so)
        return carry

    lax.fori_loop(0, nsteps // 2, pair_body, 0)
    for b in range(2):
        tile_v, lin_v, sin, so = bufs[b]
        last_i = nsteps - 2 + b
        @pl.when(active(last_i))
        def _():
            pltpu.make_async_copy(
                lin_v, lin_hbm.at[pl.ds(0, RBLK * DIM)], so
            ).wait()

    @pl.when(wid == 0)
    def _():
        pltpu.sync_copy(tail_lin, l0.at[pl.ds(0, TAIL * DIM)])
        pltpu.sync_copy(
            l0.at[pl.ds(0, TAIL * DIM)],
            lin_hbm.at[pl.ds(NFULL * RBLK * DIM, TAIL * DIM)],
        )


def _transpose_table(embeddings_t, tail_lin):
    mesh = plsc.VectorSubcoreMesh(core_axis_name="c", subcore_axis_name="s")
    return pl.kernel(
        _tr_body,
        out_type=jax.ShapeDtypeStruct((NUM_EMB * DIM,), jnp.float32),
        mesh=mesh,
        compiler_params=pltpu.CompilerParams(
            use_tc_tiling_on_sc=True, needs_layout_passes=False),
        scratch_types=[
            pltpu.VMEM((DIM, RBLK), jnp.float32),
            pltpu.VMEM((DIM, RBLK), jnp.float32),
            pltpu.VMEM((RBLK * DIM,), jnp.float32),
            pltpu.VMEM((RBLK * DIM,), jnp.float32),
            pltpu.SemaphoreType.DMA,
            pltpu.SemaphoreType.DMA,
            pltpu.SemaphoreType.DMA,
            pltpu.SemaphoreType.DMA,
        ],
    )(embeddings_t, tail_lin)


ROWS_PER_W = TOTAL // NW
G = 128
QG = 2
SG_ROWS = G * QG
NSTEP_B = ROWS_PER_W // SG_ROWS
GVREG = SG_ROWS // L


def _body(idx_hbm, table_hbm, out_hbm, idx_v, r0b, r1b, o0, o1,
          sg0, sg1, ss0, ss1):
    wid = lax.axis_index("s") * NC + lax.axis_index("c")
    wbase = wid * ROWS_PER_W
    pltpu.sync_copy(idx_hbm.at[pl.ds(wbase, ROWS_PER_W)], idx_v)
    iota = lax.broadcasted_iota(jnp.int32, (L,), 0)

    def grp(i, q):
        return wid * NSTEP_B * QG + i * QG + q

    bufs = ((r0b, o0, sg0, ss0), (r1b, o1, sg1, ss1))

    def start_gather(i, parity):
        rows_v, _, sg, _ = bufs[parity]

        @pl.when(i < NSTEP_B)
        def _():
            pltpu.async_copy(
                table_hbm.at[idx_v.at[pl.ds(i * SG_ROWS, SG_ROWS)]], rows_v, sg
            )

    def pad_scan(i):
        def scan_body(g, acc):
            return jnp.minimum(acc, idx_v[pl.ds(i * SG_ROWS + g * L, L)])

        acc = lax.fori_loop(
            0, GVREG, scan_body, jnp.full((L,), NUM_EMB, jnp.int32)
        )
        mn = acc[0]
        for j in range(1, L):
            mn = jnp.minimum(mn, acc[j])
        return mn

    def fixup(i, mn, rows_v):
        @pl.when(mn == PAD_IDX)
        def _():
            z = jnp.zeros((L,), jnp.float32)

            def fix_body(g, c):
                v = idx_v[pl.ds(i * SG_ROWS + g * L, L)]
                for r in range(L):
                    @pl.when(v[r] == PAD_IDX)
                    def _zero_row(row=g * L + r):
                        for h in range(DIM // L):
                            rows_v[row, pl.ds(h * L, L)] = z

                return c

            lax.fori_loop(0, GVREG, fix_body, 0)

    def transpose_group(rows_v, out_t):
        def sub_body(ch, carry):
            for q in range(QG):
                for u in range(2):
                    c = ch * 2 + u
                    bv = q * G + c * L + iota
                    for d_base in range(0, DIM, L):
                        skews = [
                            d_base + ((iota + k) & (L - 1)) for k in range(L)
                        ]
                        vs = [
                            plsc.load_gather(rows_v, [bv, dv]) for dv in skews
                        ]
                        for dv, v in zip(skews, vs):
                            plsc.store_scatter(
                                out_t,
                                [dv * G + iota + c * L + q * (4 * 1024)], v
                            )
            return carry

        lax.fori_loop(0, G // L // 2, sub_body, 0)

    def out_slices(i):
        res = []
        for q in range(QG):
            g = grp(i, q)
            h = g >> 5
            bb = g & 31
            for db in range(4):
                res.append((q, db, h, bb))
        return res

    def step(i, parity, rows_v, out_t, sg, ss):
        start_gather(i + 1, 1 - parity)

        @pl.when(i >= 2)
        def _():
            for q, db, h, bb in out_slices(i - 2):
                pltpu.make_async_copy(
                    out_t.at[pl.ds(q * 4096 + db * 1024, 1024)],
                    out_hbm.at[h, db, bb], ss
                ).wait()

        mn = pad_scan(i)
        pltpu.make_async_copy(
            table_hbm.at[idx_v.at[pl.ds(i * SG_ROWS, SG_ROWS)]], rows_v, sg
        ).wait()
        fixup(i, mn, rows_v)
        transpose_group(rows_v, out_t)
        for q, db, h, bb in out_slices(i):
            pltpu.async_copy(
                out_t.at[pl.ds(q * 4096 + db * 1024, 1024)],
                out_hbm.at[h, db, bb], ss
            )

    start_gather(0, 0)

    def pair_body(k, carry):
        for b in range(2):
            rows_v, out_t, sg, ss = bufs[b]
            step(2 * k + b, b, rows_v, out_t, sg, ss)
        return carry

    lax.fori_loop(0, NSTEP_B // 2, pair_body, 0)

    for b in range(2):
        rows_v, out_t, _, ss = bufs[b]
        for q, db, h, bb in out_slices(NSTEP_B - 2 + b):
            pltpu.make_async_copy(
                out_t.at[pl.ds(q * 4096 + db * 1024, 1024)],
                out_hbm.at[h, db, bb], ss
            ).wait()


def kernel(idx, embeddings, padding_mult):
    tail_lin = embeddings[NFULL * RBLK:].reshape(-1)
    lin_flat = _transpose_table(embeddings.T, tail_lin)
    lin_table = lin_flat.reshape(NUM_EMB, DIM)
    idx_flat = idx.T.reshape(-1)
    mesh = plsc.VectorSubcoreMesh(core_axis_name="c", subcore_axis_name="s")
    out5d = pl.kernel(
        _body,
        out_type=jax.ShapeDtypeStruct((200, 4, 32, 1024), jnp.float32),
        mesh=mesh,
        compiler_params=pltpu.CompilerParams(
            use_tc_tiling_on_sc=False, needs_layout_passes=False),
        scratch_types=[
            pltpu.VMEM((ROWS_PER_W,), jnp.int32),
            pltpu.VMEM((SG_ROWS, DIM), jnp.float32),
            pltpu.VMEM((SG_ROWS, DIM), jnp.float32),
            pltpu.VMEM((QG * 4 * 1024,), jnp.float32),
            pltpu.VMEM((QG * 4 * 1024,), jnp.float32),
            pltpu.SemaphoreType.DMA,
            pltpu.SemaphoreType.DMA,
            pltpu.SemaphoreType.DMA,
            pltpu.SemaphoreType.DMA,
        ],
    )(idx_flat, lin_table)
    out = (out5d.reshape(200, 4, 32, 8, 128)
           .transpose(2, 4, 0, 1, 3)
           .reshape(4096, 200, 32))
    return out


# speedup vs baseline: 3.9157x; 1.0969x over previous
import jax
import jax.numpy as jnp
from jax import lax
from jax.experimental import pallas as pl
from jax.experimental.pallas import tpu as pltpu
from jax.experimental.pallas import tpu_sc as plsc

NUM_EMB = 1000000
DIM = 32
PAD_IDX = 0
TOTAL = 4096 * 200
NC, NS, L = 2, 16, 16
NW = NC * NS

RBLK = 256
NFULL = NUM_EMB // RBLK
TAIL = NUM_EMB - NFULL * RBLK


def _tr_body(tbl_t, tail_lin, lin_hbm, t0, t1, l0, l1, sin0, sin1, so0, so1):
    wid = lax.axis_index("s") * NC + lax.axis_index("c")
    iota = lax.broadcasted_iota(jnp.int32, (L,), 0)

    def transpose_block(tile_v, lin_v, nrows):
        def sub_body(ch, carry):
            for u in range(2):
                c = ch * 2 + u
                colv = c * L + iota
                for d_base in range(0, DIM, L):
                    skews = [d_base + ((iota + k) & (L - 1)) for k in range(L)]
                    vs = [
                        plsc.load_gather(tile_v, [rowv, colv])
                        for rowv in skews
                    ]
                    for rowv, v in zip(skews, vs):
                        plsc.store_scatter(
                            lin_v, [iota * DIM + rowv + c * (L * DIM)], v
                        )
            return carry

        lax.fori_loop(0, nrows // L // 2, sub_body, 0)

    bufs = ((t0, l0, sin0, so0), (t1, l1, sin1, so1))
    nsteps = 2 * (((NFULL - 1) // NW + 1 + 1) // 2)

    def blk(i):
        return (wid + i * NW) * RBLK

    def active(i):
        return blk(i) < NFULL * RBLK

    def start_in(i, parity):
        tile_v, _, sin, _ = bufs[parity]

        @pl.when(active(i))
        def _():
            pltpu.async_copy(tbl_t.at[:, pl.ds(blk(i), RBLK)], tile_v, sin)

    def step(i, parity, tile_v, lin_v, sin, so):
        start_in(i + 1, 1 - parity)

        @pl.when((i >= 2) & active(i - 2))
        def _():
            pltpu.make_async_copy(
                lin_v, lin_hbm.at[pl.ds(0, RBLK * DIM)], so
            ).wait()

        @pl.when(active(i))
        def _():
            pltpu.make_async_copy(
                tbl_t.at[:, pl.ds(blk(i), RBLK)], tile_v, sin
            ).wait()
            transpose_block(tile_v, lin_v, RBLK)
            pltpu.async_copy(
                lin_v, lin_hbm.at[pl.ds(blk(i) * DIM, RBLK * DIM)], so
            )

    start_in(0, 0)

    def pair_body(k, carry):
        for b in range(2):
            tile_v, lin_v, sin, so = bufs[b]
            step(2 * k + b, b, tile_v, lin_v, sin, so)
        return carry

    lax.fori_loop(0, nsteps // 2, pair_body, 0)
    for b in range(2):
        tile_v, lin_v, sin, so = bufs[b]
        last_i = nsteps - 2 + b
        @pl.when(active(last_i))
        def _():
            pltpu.make_async_copy(
                lin_v, lin_hbm.at[pl.ds(0, RBLK * DIM)], so
            ).wait()

    @pl.when(wid == 0)
    def _():
        pltpu.sync_copy(tail_lin, l0.at[pl.ds(0, TAIL * DIM)])
        pltpu.sync_copy(
            l0.at[pl.ds(0, TAIL * DIM)],
            lin_hbm.at[pl.ds(NFULL * RBLK * DIM, TAIL * DIM)],
        )


def _transpose_table(embeddings_t, tail_lin):
    mesh = plsc.VectorSubcoreMesh(core_axis_name="c", subcore_axis_name="s")
    return pl.kernel(
        _tr_body,
        out_type=jax.ShapeDtypeStruct((NUM_EMB * DIM,), jnp.float32),
        mesh=mesh,
        compiler_params=pltpu.CompilerParams(
            use_tc_tiling_on_sc=True, needs_layout_passes=False),
        scratch_types=[
            pltpu.VMEM((DIM, RBLK), jnp.float32),
            pltpu.VMEM((DIM, RBLK), jnp.float32),
            pltpu.VMEM((RBLK * DIM,), jnp.float32),
            pltpu.VMEM((RBLK * DIM,), jnp.float32),
            pltpu.SemaphoreType.DMA,
            pltpu.SemaphoreType.DMA,
            pltpu.SemaphoreType.DMA,
            pltpu.SemaphoreType.DMA,
        ],
    )(embeddings_t, tail_lin)


ROWS_PER_W = TOTAL // NW
G = 128
QG = 4
SG_ROWS = G * QG
NSTEP_B = ROWS_PER_W // SG_ROWS
GVREG = SG_ROWS // L


def _body(idx_hbm, table_hbm, out_hbm, idx_v, r0b, r1b, o0, o1,
          sg0, sg1, ss0, ss1):
    wid = lax.axis_index("s") * NC + lax.axis_index("c")
    wbase = wid * ROWS_PER_W
    pltpu.sync_copy(idx_hbm.at[pl.ds(wbase, ROWS_PER_W)], idx_v)
    iota = lax.broadcasted_iota(jnp.int32, (L,), 0)

    def grp(i, q):
        return wid * NSTEP_B * QG + i * QG + q

    bufs = ((r0b, o0, sg0, ss0), (r1b, o1, sg1, ss1))

    def start_gather(i, parity):
        rows_v, _, sg, _ = bufs[parity]

        @pl.when(i < NSTEP_B)
        def _():
            pltpu.async_copy(
                table_hbm.at[idx_v.at[pl.ds(i * SG_ROWS, SG_ROWS)]], rows_v, sg
            )

    def pad_scan(i):
        def scan_body(g, acc):
            return jnp.minimum(acc, idx_v[pl.ds(i * SG_ROWS + g * L, L)])

        acc = lax.fori_loop(
            0, GVREG, scan_body, jnp.full((L,), NUM_EMB, jnp.int32)
        )
        mn = acc[0]
        for j in range(1, L):
            mn = jnp.minimum(mn, acc[j])
        return mn

    def fixup(i, mn, rows_v):
        @pl.when(mn == PAD_IDX)
        def _():
            z = jnp.zeros((L,), jnp.float32)

            def fix_body(g, c):
                v = idx_v[pl.ds(i * SG_ROWS + g * L, L)]
                for r in range(L):
                    @pl.when(v[r] == PAD_IDX)
                    def _zero_row(row=g * L + r):
                        for h in range(DIM // L):
                            rows_v[row, pl.ds(h * L, L)] = z

                return c

            lax.fori_loop(0, GVREG, fix_body, 0)

    def transpose_group(rows_v, out_t):
        def sub_body(ch, carry):
            for q in range(QG):
                for u in range(2):
                    c = ch * 2 + u
                    bv = q * G + c * L + iota
                    for d_base in range(0, DIM, L):
                        skews = [
                            d_base + ((iota + k) & (L - 1)) for k in range(L)
                        ]
                        vs = [
                            plsc.load_gather(rows_v, [bv, dv]) for dv in skews
                        ]
                        for dv, v in zip(skews, vs):
                            plsc.store_scatter(
                                out_t,
                                [dv * G + iota + c * L + q * (4 * 1024)], v
                            )
            return carry

        lax.fori_loop(0, G // L // 2, sub_body, 0)

    def out_slices(i):
        res = []
        for q in range(QG):
            g = grp(i, q)
            h = g >> 5
            bb = g & 31
            for db in range(4):
                res.append((q, db, h, bb))
        return res

    def step(i, parity, rows_v, out_t, sg, ss):
        start_gather(i + 1, 1 - parity)

        @pl.when(i >= 2)
        def _():
            for q, db, h, bb in out_slices(i - 2):
                pltpu.make_async_copy(
                    out_t.at[pl.ds(q * 4096 + db * 1024, 1024)],
                    out_hbm.at[h, db, bb], ss
                ).wait()

        mn = pad_scan(i)
        pltpu.make_async_copy(
            table_hbm.at[idx_v.at[pl.ds(i * SG_ROWS, SG_ROWS)]], rows_v, sg
        ).wait()
        fixup(i, mn, rows_v)
        transpose_group(rows_v, out_t)
        for q, db, h, bb in out_slices(i):
            pltpu.async_copy(
                out_t.at[pl.ds(q * 4096 + db * 1024, 1024)],
                out_hbm.at[h, db, bb], ss
            )

    start_gather(0, 0)

    def pair_body(k, carry):
        for b in range(2):
            rows_v, out_t, sg, ss = bufs[b]
            step(2 * k + b, b, rows_v, out_t, sg, ss)
        return carry

    lax.fori_loop(0, NSTEP_B // 2, pair_body, 0)

    for b in range(2):
        rows_v, out_t, _, ss = bufs[b]
        for q, db, h, bb in out_slices(NSTEP_B - 2 + b):
            pltpu.make_async_copy(
                out_t.at[pl.ds(q * 4096 + db * 1024, 1024)],
                out_hbm.at[h, db, bb], ss
            ).wait()


def kernel(idx, embeddings, padding_mult):
    tail_lin = embeddings[NFULL * RBLK:].reshape(-1)
    lin_flat = _transpose_table(embeddings.T, tail_lin)
    lin_table = lin_flat.reshape(NUM_EMB, DIM)
    idx_flat = idx.T.reshape(-1)
    mesh = plsc.VectorSubcoreMesh(core_axis_name="c", subcore_axis_name="s")
    out5d = pl.kernel(
        _body,
        out_type=jax.ShapeDtypeStruct((200, 4, 32, 1024), jnp.float32),
        mesh=mesh,
        compiler_params=pltpu.CompilerParams(
            use_tc_tiling_on_sc=False, needs_layout_passes=False),
        scratch_types=[
            pltpu.VMEM((ROWS_PER_W,), jnp.int32),
            pltpu.VMEM((SG_ROWS, DIM), jnp.float32),
            pltpu.VMEM((SG_ROWS, DIM), jnp.float32),
            pltpu.VMEM((QG * 4 * 1024,), jnp.float32),
            pltpu.VMEM((QG * 4 * 1024,), jnp.float32),
            pltpu.SemaphoreType.DMA,
            pltpu.SemaphoreType.DMA,
            pltpu.SemaphoreType.DMA,
            pltpu.SemaphoreType.DMA,
        ],
    )(idx_flat, lin_table)
    out = (out5d.reshape(200, 4, 32, 8, 128)
           .transpose(2, 4, 0, 1, 3)
           .reshape(4096, 200, 32))
    return out
